# baseline (speedup 1.0000x reference)
import functools
import math

import jax
import jax.numpy as jnp
from jax import lax
from jax.experimental import pallas as pl
from jax.experimental.pallas import tpu as pltpu
from jax.experimental.pallas import tpu_sc as plsc

B = 1024
V = 100000
M = 0.5
COS_M = math.cos(M)
SIN_M = math.sin(M)

_TCR = 512
_SCR = B - _TCR


_NC = 2
_NS = 16
_NW = _NC * _NS
_RPT = _SCR // _NW
_G = 8
_NGRP = _RPT // _G
_CW = 6144
_NCH = 16
_SCCOLS = _CW * _NCH
_TAILW = V - _SCCOLS
_UNR = 16
_ROWIT = _CW // (16 * _UNR)


def _row_sums(buf, accs, cvecs, rels):
    lane = lax.iota(jnp.int32, 16)
    outa, outc = [], []
    for r in range(_G):
        def body(i, ac, r=r):
            a, c = ac
            base = i * (16 * _UNR)
            for u in range(_UNR):
                o = base + u * 16
                v = buf[r, pl.ds(o, 16)]
                a = a + jnp.exp(v)
                c = jnp.where(lane == rels[r] - o, v, c)
            return (a, c)
        a, c = lax.fori_loop(0, _ROWIT, body, (accs[r], cvecs[r]))
        outa.append(a)
        outc.append(c)
    return tuple(outa), tuple(outc)


@functools.cache
def _build_sc_sumexp():
    mesh = plsc.VectorSubcoreMesh(core_axis_name="c", subcore_axis_name="s", num_cores=2)

    @functools.partial(
        pl.kernel,
        mesh=mesh,
        out_type=(
            jax.ShapeDtypeStruct((_SCR, 16), jnp.float32),
            jax.ShapeDtypeStruct((_SCR, 16), jnp.float32),
        ),
        scratch_types=[
            pltpu.VMEM((_RPT, 16), jnp.int32),
            pltpu.VMEM((_RPT, 16), jnp.float32),
            pltpu.VMEM((_RPT, 16), jnp.float32),
            pltpu.VMEM((_G, _CW), jnp.float32),
            pltpu.VMEM((_G, _CW), jnp.float32),
            pltpu.SemaphoreType.DMA,
            pltpu.SemaphoreType.DMA,
        ],
    )
    def sc_kernel(x_hbm, tgtb_hbm, s16_out, c16_out,
                  tgtb_v, c16, srow, buf_a, buf_b, sem_a, sem_b):
        wid = lax.axis_index("s") * _NC + lax.axis_index("c")
        base = wid * _RPT
        pltpu.sync_copy(tgtb_hbm.at[pl.ds(_TCR + base, _RPT)], tgtb_v)
        zero16 = jnp.zeros((16,), jnp.float32)

        def start(rows0, ch, buf, sem):
            pltpu.make_async_copy(
                x_hbm.at[pl.ds(rows0, _G), pl.ds(ch * _CW, _CW)],
                buf, sem).start()

        def wait(rows0, ch, buf, sem):
            pltpu.make_async_copy(
                x_hbm.at[pl.ds(rows0, _G), pl.ds(ch * _CW, _CW)],
                buf, sem).wait()

        for g in range(_NGRP):
            rows0 = _TCR + base + g * _G
            tvecs = [tgtb_v[g * _G + r] for r in range(_G)]
            start(rows0, 0, buf_a, sem_a)

            def pair_body(p, carry, rows0=rows0, tvecs=tvecs):
                accs, cvecs = carry
                off_a = 2 * p * _CW
                off_b = (2 * p + 1) * _CW
                start(rows0, 2 * p + 1, buf_b, sem_b)
                wait(rows0, 2 * p, buf_a, sem_a)
                rels_a = [tvecs[r] - off_a for r in range(_G)]
                accs, cvecs = _row_sums(buf_a, accs, cvecs, rels_a)

                @pl.when(p + 1 < _NCH // 2)
                def _next():
                    start(rows0, 2 * p + 2, buf_a, sem_a)

                wait(rows0, 2 * p + 1, buf_b, sem_b)
                rels_b = [tvecs[r] - off_b for r in range(_G)]
                accs, cvecs = _row_sums(buf_b, accs, cvecs, rels_b)
                return (accs, cvecs)

            accs, cvecs = lax.fori_loop(
                0, _NCH // 2, pair_body,
                (tuple(zero16 for _ in range(_G)),
                 tuple(zero16 for _ in range(_G))))

            for r in range(_G):
                srow[g * _G + r] = accs[r]
                c16[g * _G + r] = cvecs[r]

        pltpu.sync_copy(srow, s16_out.at[pl.ds(base, _RPT)])
        pltpu.sync_copy(c16, c16_out.at[pl.ds(base, _RPT)])

    return sc_kernel


_RB = 8
_NRB = _TCR // _RB
_CH = 2048
_NFULL = V // _CH
_T0 = _NFULL * _CH
_T128 = ((V - _T0) // 128) * 128
_NBUF = 2
_SEG = [(0, 25088), (25088, 25088), (50176, 25088), (75264, V - 75264)]


def _tc_start(x_hbm, buf, sems, step, slot):
    for k, (off, ln) in enumerate(_SEG):
        pltpu.make_async_copy(
            x_hbm.at[pl.ds(step * _RB, _RB), pl.ds(off, ln)],
            buf.at[slot, :, pl.ds(off, ln)],
            sems.at[slot, k],
        ).start()


def _tc_wait(x_hbm, buf, sems, step, slot):
    for k, (off, ln) in enumerate(_SEG):
        pltpu.make_async_copy(
            x_hbm.at[pl.ds(step * _RB, _RB), pl.ds(off, ln)],
            buf.at[slot, :, pl.ds(off, ln)],
            sems.at[slot, k],
        ).wait()


def _tc_stream_body(x_hbm, tgt_ref, s_ref, c_ref, buf, sems):
    i = pl.program_id(0)
    slot = lax.rem(i, _NBUF)

    @pl.when(i == 0)
    def _prime():
        _tc_start(x_hbm, buf, sems, 0, 0)

    @pl.when(i + 1 < _NRB)
    def _prefetch():
        _tc_start(x_hbm, buf, sems, i + 1, lax.rem(i + 1, _NBUF))

    _tc_wait(x_hbm, buf, sems, i, slot)

    x = buf[slot]
    acc = jnp.exp(x[:, 0:_CH])
    for k in range(1, _NFULL):
        acc += jnp.exp(x[:, k * _CH:(k + 1) * _CH])
    rowsum = jnp.sum(acc, axis=1, keepdims=True)
    rowsum += jnp.sum(jnp.exp(x[:, _T0:_T0 + _T128]), axis=1, keepdims=True)
    rowsum += jnp.sum(jnp.exp(x[:, _T0 + _T128:V]), axis=1, keepdims=True)
    s_ref[...] = rowsum

    l128 = lax.broadcasted_iota(jnp.int32, (1, 128), 1)
    cs = []
    for r in range(_RB):
        t = tgt_ref[r, 0]
        ta = pl.multiple_of((t // 128) * 128, 128)
        v = buf[slot, r:r + 1, pl.ds(ta, 128)]
        cs.append(jnp.sum(jnp.where(l128 == t - ta, v, 0.0),
                          axis=1, keepdims=True))
    c_ref[...] = jnp.concatenate(cs, axis=0)


def _tc_stream(inp, tgt):
    return pl.pallas_call(
        _tc_stream_body,
        grid=(_NRB,),
        in_specs=[
            pl.BlockSpec(memory_space=pl.ANY),
            pl.BlockSpec((_RB, 1), lambda i: (i, 0),
                         memory_space=pltpu.SMEM),
        ],
        out_specs=[
            pl.BlockSpec((_RB, 1), lambda i: (i, 0)),
            pl.BlockSpec((_RB, 1), lambda i: (i, 0)),
        ],
        out_shape=[
            jax.ShapeDtypeStruct((_TCR, 1), jnp.float32),
            jax.ShapeDtypeStruct((_TCR, 1), jnp.float32),
        ],
        scratch_shapes=[
            pltpu.VMEM((_NBUF, _RB, V), jnp.float32),
            pltpu.SemaphoreType.DMA((_NBUF, len(_SEG))),
        ],
    )(inp, tgt)


_TB = 2048


def _combine_body(s16_ref, c16_ref, stc_ref, ctc_ref, tgt_ref, xtail_ref,
                  out_ref):
    xt = xtail_ref[...]
    colid = lax.broadcasted_iota(jnp.int32, (_SCR, _TB), 1)
    valid = colid < _TAILW
    e = jnp.where(valid, jnp.exp(xt), 0.0)
    tail_sum = jnp.sum(e, axis=1, keepdims=True)

    trel = tgt_ref[...] - _SCCOLS
    hit = (colid == trel) & valid
    c_tail = jnp.sum(jnp.where(hit, xt, 0.0), axis=1, keepdims=True)
    c_sc = jnp.sum(c16_ref[...], axis=1, keepdims=True)
    c_sc = jnp.where(trel >= 0, c_tail, c_sc)
    s_sc = jnp.sum(s16_ref[...], axis=1, keepdims=True) + tail_sum

    def nll_sum(s, c):
        sin_t = jnp.sqrt(jnp.maximum(1.0 - c * c, 0.0))
        new_cos = c * COS_M - sin_t * SIN_M
        stot = s - jnp.exp(c) + jnp.exp(new_cos)
        return jnp.sum(jnp.log(stot) - new_cos)

    out_ref[0, 0] = (nll_sum(s_sc, c_sc)
                     + nll_sum(stc_ref[...], ctc_ref[...])) / B


def _tc_combine(s16, c16, s_tc, c_tc, tgt_sc, inp):
    return pl.pallas_call(
        _combine_body,
        grid=(1,),
        in_specs=[
            pl.BlockSpec((_SCR, 16), lambda i: (0, 0)),
            pl.BlockSpec((_SCR, 16), lambda i: (0, 0)),
            pl.BlockSpec((_TCR, 1), lambda i: (0, 0)),
            pl.BlockSpec((_TCR, 1), lambda i: (0, 0)),
            pl.BlockSpec((_SCR, 1), lambda i: (0, 0)),
            pl.BlockSpec((_SCR, _TB), lambda i: (_TCR // _SCR, _SCCOLS // _TB)),
        ],
        out_specs=pl.BlockSpec(memory_space=pltpu.SMEM),
        out_shape=jax.ShapeDtypeStruct((1, 1), jnp.float32),
    )(s16, c16, s_tc, c_tc, tgt_sc, inp)


def kernel(input, target):
    tgt = target.astype(jnp.int32).reshape(B, 1)
    tgt_b = jnp.broadcast_to(tgt, (B, 16))
    s16, c16 = _build_sc_sumexp()(input, tgt_b)
    s_tc, c_tc = _tc_stream(input, tgt[:_TCR])
    out = _tc_combine(s16, c16, s_tc, c_tc, tgt[_TCR:], input)
    return out[0, 0]


# speedup vs baseline: 2.2991x; 2.2991x over previous
import functools
import math

import jax
import jax.numpy as jnp
from jax import lax
from jax.experimental import pallas as pl
from jax.experimental.pallas import tpu as pltpu
from jax.experimental.pallas import tpu_sc as plsc

B = 1024
V = 100000
M = 0.5
COS_M = math.cos(M)
SIN_M = math.sin(M)


_NC = 2
_NS = 16
_NW = _NC * _NS
_STRIPE = 1280
_SCV = _STRIPE * _NW
_CR = 40
_NCHK = _STRIPE // _CR
_NCOL = B // 16


@functools.cache
def _build_sc_part():
    mesh = plsc.VectorSubcoreMesh(core_axis_name="c", subcore_axis_name="s")

    @functools.partial(
        pl.kernel,
        mesh=mesh,
        out_type=(
            jax.ShapeDtypeStruct((_NW, B), jnp.float32),
            jax.ShapeDtypeStruct((_NW, B), jnp.float32),
        ),
        scratch_types=[
            pltpu.VMEM((B,), jnp.int32),
            pltpu.VMEM((B,), jnp.float32),
            pltpu.VMEM((B,), jnp.float32),
            pltpu.VMEM((_CR, B), jnp.float32),
            pltpu.VMEM((_CR, B), jnp.float32),
            pltpu.SemaphoreType.DMA,
            pltpu.SemaphoreType.DMA,
        ],
    )
    def sc_kernel(xt_hbm, tgt_hbm, s_out, c_out,
                  tgt_v, acc_v, c_v, buf_a, buf_b, sem_a, sem_b):
        wid = lax.axis_index("s") * _NC + lax.axis_index("c")
        stripe0 = wid * _STRIPE
        pltpu.sync_copy(tgt_hbm, tgt_v)
        zero16 = jnp.zeros((16,), jnp.float32)

        def zbody(z, carry):
            acc_v[pl.ds(z * 16, 16)] = zero16
            c_v[pl.ds(z * 16, 16)] = zero16
            return carry

        lax.fori_loop(0, _NCOL, zbody, 0)

        def start(ch, buf, sem):
            pltpu.make_async_copy(
                xt_hbm.at[pl.ds(stripe0 + ch * _CR, _CR), :], buf, sem).start()

        def wait(ch, buf, sem):
            pltpu.make_async_copy(
                xt_hbm.at[pl.ds(stripe0 + ch * _CR, _CR), :], buf, sem).wait()

        def process(buf, gbase):
            def jbody(j, carry):
                js = pl.ds(j * 16, 16)
                trel = tgt_v[js] - gbase
                a = acc_v[js]
                c = c_v[js]
                for i in range(_CR):
                    v = buf[i, js]
                    a = a + jnp.exp(v)
                    c = jnp.where(trel == i, v, c)
                acc_v[js] = a
                c_v[js] = c
                return carry
            lax.fori_loop(0, _NCOL, jbody, 0)

        start(0, buf_a, sem_a)

        def pair_body(p, carry):
            start(2 * p + 1, buf_b, sem_b)
            wait(2 * p, buf_a, sem_a)
            process(buf_a, stripe0 + 2 * p * _CR)

            @pl.when(p + 1 < _NCHK // 2)
            def _next():
                start(2 * p + 2, buf_a, sem_a)

            wait(2 * p + 1, buf_b, sem_b)
            process(buf_b, stripe0 + (2 * p + 1) * _CR)
            return carry

        lax.fori_loop(0, _NCHK // 2, pair_body, 0)

        pltpu.sync_copy(acc_v, s_out.at[wid])
        pltpu.sync_copy(c_v, c_out.at[wid])

    return sc_kernel


_VB = 2048
_VB0 = _SCV // _VB
_NBT = -(-(V - _SCV) // _VB)


def _tc_body(xt_ref, tgt_ref, sp_ref, cp_ref, out_ref, acc_ref, cacc_ref):
    i = pl.program_id(0)

    @pl.when(i == 0)
    def _init():
        acc_ref[...] = jnp.zeros_like(acc_ref)
        cacc_ref[...] = jnp.zeros_like(cacc_ref)

    rowbase = (_VB0 + i) * _VB
    tvec = tgt_ref[...]
    acc = acc_ref[...]
    cacc = cacc_ref[...]
    for k in range(_VB // 8):
        xs = xt_ref[k * 8:(k + 1) * 8, :]
        rid = (lax.broadcasted_iota(jnp.int32, (8, B), 0)
               + (rowbase + k * 8))
        acc += jnp.where(rid < V, jnp.exp(xs), 0.0)
        cacc += jnp.where(rid == tvec, xs, 0.0)
    acc_ref[...] = acc
    cacc_ref[...] = cacc

    @pl.when(i == _NBT - 1)
    def _finish():
        s = jnp.sum(acc_ref[...], axis=0, keepdims=True)
        c = jnp.sum(cacc_ref[...], axis=0, keepdims=True)
        s += jnp.sum(sp_ref[...], axis=0, keepdims=True)
        c += jnp.sum(cp_ref[...], axis=0, keepdims=True)
        sin_t = jnp.sqrt(jnp.maximum(1.0 - c * c, 0.0))
        new_cos = c * COS_M - sin_t * SIN_M
        stot = s - jnp.exp(c) + jnp.exp(new_cos)
        nll = jnp.log(stot) - new_cos
        out_ref[0, 0] = jnp.sum(nll) / B


def _tc_loss(xt, tgt, s_part, c_part):
    return pl.pallas_call(
        _tc_body,
        grid=(_NBT,),
        in_specs=[
            pl.BlockSpec((_VB, B), lambda i: (_VB0 + i, 0)),
            pl.BlockSpec((1, B), lambda i: (0, 0)),
            pl.BlockSpec((_NW, B), lambda i: (0, 0)),
            pl.BlockSpec((_NW, B), lambda i: (0, 0)),
        ],
        out_specs=pl.BlockSpec(memory_space=pltpu.SMEM),
        out_shape=jax.ShapeDtypeStruct((1, 1), jnp.float32),
        scratch_shapes=[
            pltpu.VMEM((8, B), jnp.float32),
            pltpu.VMEM((8, B), jnp.float32),
        ],
    )(xt, tgt, s_part, c_part)


def kernel(input, target):
    xt = input.T
    tgt = target.astype(jnp.int32)
    s_part, c_part = _build_sc_part()(xt, tgt)
    out = _tc_loss(xt, tgt.reshape(1, B), s_part, c_part)
    return out[0, 0]
